# baseline (speedup 1.0000x reference)
import jax
import jax.numpy as jnp
from jax.experimental import pallas as pl
from jax.experimental.pallas import tpu as pltpu

_K = 64
_ROWS_PER_BLOCK = 16


def _topk_mask_body(x_ref, o_ref, cut_ref):
    x = x_ref[...]
    i = jax.lax.bitcast_convert_type(x, jnp.int32)
    key = jnp.where(i >= 0, i, jnp.bitwise_xor(jnp.bitwise_not(i), jnp.int32(-(2**31))))
    nrows = x.shape[0]
    kmin = jnp.min(key, axis=1, keepdims=True)
    kmax = jnp.max(key, axis=1, keepdims=True)
    ones_b = jnp.ones((x.shape[1], 1), jnp.bfloat16)

    def cnt_ge(t):
        ind = jnp.where(key >= t, 1.0, 0.0).astype(jnp.bfloat16)
        return jax.lax.dot_general(
            ind, ones_b, (((1,), (0,)), ((), ())),
            preferred_element_type=jnp.float32)

    def cond(carry):
        lo, hi, found, thr = carry
        return jnp.any((found == 0) & ((hi - 1) > lo))

    def body(_, carry):
        lo, hi, found, thr = carry
        mid = (lo >> 1) + (hi >> 1) + (lo & hi & 1)
        cnt = cnt_ge(mid)
        hit = cnt == float(_K)
        ge = cnt >= float(_K)
        thr = jnp.where(hit & (found == 0), mid, thr)
        found = found | hit.astype(jnp.int32)
        lo = jnp.where(ge, mid, lo)
        hi = jnp.where(ge, hi, mid)
        return lo, hi, found, thr

    carry0 = (
        kmin,
        kmax + 1,
        jnp.zeros((nrows, 1), jnp.int32),
        kmin,
    )
    lo, hi, found, thr = jax.lax.fori_loop(0, 32, body, carry0)
    thr = jnp.where(found == 1, thr, lo)

    n_ge = cnt_ge(thr)
    col = jax.lax.broadcasted_iota(jnp.int32, x.shape, 1)
    cut_ref[...] = jnp.full((nrows, 1), jnp.iinfo(jnp.int32).max, jnp.int32)

    @pl.when(jnp.any(n_ge > float(_K)))
    def _():
        extra = n_ge.astype(jnp.int32) - _K
        tcol = jnp.where(key == thr, col, -1)
        cut = jnp.full((nrows, 1), jnp.iinfo(jnp.int32).max, jnp.int32)
        for _ in range(4):
            hi_col = jnp.max(jnp.where(tcol < cut, tcol, -1), axis=1, keepdims=True)
            cut = jnp.where(extra > 0, hi_col, cut)
            extra = jnp.maximum(extra - 1, 0)
        cut_ref[...] = cut

    cut = cut_ref[...]
    keep = (key > thr) | ((key == thr) & (col < cut))
    o_ref[...] = jnp.where(keep, jnp.maximum(x, 0.0), 0.0)


def kernel(x):
    m, n = x.shape
    grid = (m // _ROWS_PER_BLOCK,)
    return pl.pallas_call(
        _topk_mask_body,
        grid=grid,
        in_specs=[pl.BlockSpec((_ROWS_PER_BLOCK, n), lambda r: (r, 0))],
        out_specs=pl.BlockSpec((_ROWS_PER_BLOCK, n), lambda r: (r, 0)),
        out_shape=jax.ShapeDtypeStruct((m, n), x.dtype),
        scratch_shapes=[pltpu.VMEM((_ROWS_PER_BLOCK, 1), jnp.int32)],
        compiler_params=pltpu.CompilerParams(
            dimension_semantics=("arbitrary",),
        ),
    )(x)


# speedup vs baseline: 3.9764x; 3.9764x over previous
import jax
import jax.numpy as jnp
from jax.experimental import pallas as pl
from jax.experimental.pallas import tpu as pltpu

_K = 64
_ROWS_PER_BLOCK = 16


def _topk_mask_body(x_ref, o_ref, cut_ref):
    x = x_ref[...]
    i = jax.lax.bitcast_convert_type(x, jnp.int32)
    key = jnp.where(i >= 0, i, jnp.bitwise_xor(jnp.bitwise_not(i), jnp.int32(-(2**31))))
    nrows = x.shape[0]
    kmin = jnp.min(key, axis=1, keepdims=True)
    kmax = jnp.max(key, axis=1, keepdims=True)
    def cnt_ge(t):
        return jnp.sum(jnp.where(key >= t, 1.0, 0.0), axis=1, keepdims=True)

    def cond(carry):
        lo, hi, found, thr = carry
        return jnp.any((found == 0) & ((hi - 1) > lo))

    def body(carry):
        lo, hi, found, thr = carry
        mid = (lo >> 1) + (hi >> 1) + (lo & hi & 1)
        cnt = cnt_ge(mid)
        hit = cnt == float(_K)
        ge = cnt >= float(_K)
        thr = jnp.where(hit & (found == 0), mid, thr)
        found = found | hit.astype(jnp.int32)
        lo = jnp.where(ge, mid, lo)
        hi = jnp.where(ge, hi, mid)
        return lo, hi, found, thr

    carry0 = (
        kmin,
        kmax + 1,
        jnp.zeros((nrows, 1), jnp.int32),
        kmin,
    )
    lo, hi, found, thr = jax.lax.while_loop(cond, body, carry0)
    thr = jnp.where(found == 1, thr, lo)

    n_ge = cnt_ge(thr)
    col = jax.lax.broadcasted_iota(jnp.int32, x.shape, 1)
    cut_ref[...] = jnp.full((nrows, 1), jnp.iinfo(jnp.int32).max, jnp.int32)

    @pl.when(jnp.any(n_ge > float(_K)))
    def _():
        extra = n_ge.astype(jnp.int32) - _K
        tcol = jnp.where(key == thr, col, -1)
        cut = jnp.full((nrows, 1), jnp.iinfo(jnp.int32).max, jnp.int32)
        for _ in range(4):
            hi_col = jnp.max(jnp.where(tcol < cut, tcol, -1), axis=1, keepdims=True)
            cut = jnp.where(extra > 0, hi_col, cut)
            extra = jnp.maximum(extra - 1, 0)
        cut_ref[...] = cut

    cut = cut_ref[...]
    keep = (key > thr) | ((key == thr) & (col < cut))
    o_ref[...] = jnp.where(keep, jnp.maximum(x, 0.0), 0.0)


def kernel(x):
    m, n = x.shape
    grid = (m // _ROWS_PER_BLOCK,)
    return pl.pallas_call(
        _topk_mask_body,
        grid=grid,
        in_specs=[pl.BlockSpec((_ROWS_PER_BLOCK, n), lambda r: (r, 0))],
        out_specs=pl.BlockSpec((_ROWS_PER_BLOCK, n), lambda r: (r, 0)),
        out_shape=jax.ShapeDtypeStruct((m, n), x.dtype),
        scratch_shapes=[pltpu.VMEM((_ROWS_PER_BLOCK, 1), jnp.int32)],
        compiler_params=pltpu.CompilerParams(
            dimension_semantics=("arbitrary",),
        ),
    )(x)


# speedup vs baseline: 4.4408x; 1.1168x over previous
import jax
import jax.numpy as jnp
from jax.experimental import pallas as pl
from jax.experimental.pallas import tpu as pltpu

_K = 64
_ROWS_PER_BLOCK = 16
_LANES = 128


def _topk_mask_body(x_ref, o_ref):
    x = x_ref[...]
    i = jax.lax.bitcast_convert_type(x, jnp.int32)
    key = jnp.where(i >= 0, i, jnp.bitwise_xor(jnp.bitwise_not(i), jnp.int32(-(2**31))))
    nrows, ncols = x.shape

    gmax = jnp.max(key.reshape(nrows, ncols // _LANES, _LANES), axis=1)
    kmax = jnp.max(gmax, axis=1, keepdims=True)

    def small_body(_, carry):
        glo, ghi = carry
        gmid = (glo >> 1) + (ghi >> 1) + (glo & ghi & 1)
        gcnt = jnp.sum(jnp.where(gmax >= gmid, 1.0, 0.0), axis=1, keepdims=True)
        ge = gcnt >= _K
        return jnp.where(ge, gmid, glo), jnp.where(ge, ghi, gmid)

    gmin = jnp.min(gmax, axis=1, keepdims=True)
    m64, _ = jax.lax.fori_loop(0, 32, small_body, (gmin, kmax + 1))

    def cnt_ge(t):
        return jnp.sum(jnp.where(key >= t, 1.0, 0.0), axis=1, keepdims=True)

    def cond(carry):
        lo, hi, found, thr, cnt_lo = carry
        return jnp.any((found == 0) & ((hi - 1) > lo))

    def body(carry):
        lo, hi, found, thr, cnt_lo = carry
        mid = (lo >> 1) + (hi >> 1) + (lo & hi & 1)
        cnt = cnt_ge(mid)
        hit = cnt == float(_K)
        ge = cnt >= float(_K)
        thr = jnp.where(hit & (found == 0), mid, thr)
        found = found | hit.astype(jnp.int32)
        lo = jnp.where(ge, mid, lo)
        hi = jnp.where(ge, hi, mid)
        cnt_lo = jnp.where(ge, cnt, cnt_lo)
        return lo, hi, found, thr, cnt_lo

    carry0 = (
        m64,
        kmax + 1,
        jnp.zeros((nrows, 1), jnp.int32),
        m64,
        cnt_ge(m64),
    )
    lo, hi, found, thr, cnt_lo = jax.lax.while_loop(cond, body, carry0)
    is_found = found == 1
    thr = jnp.where(is_found, thr, lo)
    n_ge = jnp.where(is_found, float(_K), cnt_lo)
    overflow = jnp.any(n_ge > float(_K))

    @pl.when(jnp.logical_not(overflow))
    def _():
        o_ref[...] = jnp.where(key >= thr, jnp.maximum(x, 0.0), 0.0)

    @pl.when(overflow)
    def _():
        col = jax.lax.broadcasted_iota(jnp.int32, x.shape, 1)
        extra = n_ge.astype(jnp.int32) - _K
        tcol = jnp.where(key == thr, col, -1)
        cut = jnp.full((nrows, 1), jnp.iinfo(jnp.int32).max, jnp.int32)
        for _ in range(4):
            hi_col = jnp.max(jnp.where(tcol < cut, tcol, -1), axis=1, keepdims=True)
            cut = jnp.where(extra > 0, hi_col, cut)
            extra = jnp.maximum(extra - 1, 0)
        keep = (key > thr) | ((key == thr) & (col < cut))
        o_ref[...] = jnp.where(keep, jnp.maximum(x, 0.0), 0.0)


def kernel(x):
    m, n = x.shape
    grid = (m // _ROWS_PER_BLOCK,)
    return pl.pallas_call(
        _topk_mask_body,
        grid=grid,
        in_specs=[pl.BlockSpec((_ROWS_PER_BLOCK, n), lambda r: (r, 0))],
        out_specs=pl.BlockSpec((_ROWS_PER_BLOCK, n), lambda r: (r, 0)),
        out_shape=jax.ShapeDtypeStruct((m, n), x.dtype),
        compiler_params=pltpu.CompilerParams(
            dimension_semantics=("arbitrary",),
        ),
    )(x)


# speedup vs baseline: 6.6108x; 1.4886x over previous
import jax
import jax.numpy as jnp
from jax.experimental import pallas as pl
from jax.experimental.pallas import tpu as pltpu

_K = 64
_ROWS_PER_BLOCK = 16
_LANES = 128
_TOP_PER_LANE = 8
_KEY_MIN = jnp.iinfo(jnp.int32).min


def _topk_mask_body(x_ref, o_ref):
    x = x_ref[...]
    i = jax.lax.bitcast_convert_type(x, jnp.int32)
    key = jnp.where(i >= 0, i, jnp.bitwise_xor(jnp.bitwise_not(i), jnp.int32(-(2**31))))
    nrows, ncols = x.shape
    nchunks = ncols // _LANES

    chunks = [key[:, c * _LANES:(c + 1) * _LANES] for c in range(nchunks)]

    def tree_max(vals):
        while len(vals) > 1:
            nxt = [jnp.maximum(vals[j], vals[j + 1]) for j in range(0, len(vals) - 1, 2)]
            if len(vals) % 2:
                nxt.append(vals[-1])
            vals = nxt
        return vals[0]

    tops = [tree_max(chunks)]
    for _ in range(_TOP_PER_LANE - 1):
        prev = tops[-1]
        tops.append(tree_max([jnp.where(c < prev, c, _KEY_MIN) for c in chunks]))
    cand = jnp.concatenate(tops, axis=1)
    kmax = jnp.max(tops[0], axis=1, keepdims=True)

    def small_body(_, carry):
        glo, ghi = carry
        gmid = (glo >> 1) + (ghi >> 1) + (glo & ghi & 1)
        gcnt = jnp.sum(jnp.where(cand >= gmid, 1.0, 0.0), axis=1, keepdims=True)
        ge = gcnt >= float(_K)
        return jnp.where(ge, gmid, glo), jnp.where(ge, ghi, gmid)

    cmin = jnp.min(cand, axis=1, keepdims=True)
    tstar, _ = jax.lax.fori_loop(0, 32, small_body, (cmin, kmax + 1))

    def cnt_ge(t):
        return jnp.sum(jnp.where(key >= t, 1.0, 0.0), axis=1, keepdims=True)

    def cond(carry):
        lo, hi, found, thr, cnt_lo = carry
        return jnp.any((found == 0) & ((hi - 1) > lo))

    def body(carry):
        lo, hi, found, thr, cnt_lo = carry
        mid = (lo >> 1) + (hi >> 1) + (lo & hi & 1)
        cnt = cnt_ge(mid)
        hit = cnt == float(_K)
        ge = cnt >= float(_K)
        thr = jnp.where(hit & (found == 0), mid, thr)
        found = found | hit.astype(jnp.int32)
        lo = jnp.where(ge, mid, lo)
        hi = jnp.where(ge, hi, mid)
        cnt_lo = jnp.where(ge, cnt, cnt_lo)
        return lo, hi, found, thr, cnt_lo

    cnt0 = cnt_ge(tstar)
    found0 = (cnt0 == float(_K)).astype(jnp.int32)
    carry0 = (tstar, kmax + 1, found0, tstar, cnt0)
    lo, hi, found, thr, cnt_lo = jax.lax.while_loop(cond, body, carry0)
    is_found = found == 1
    thr = jnp.where(is_found, thr, lo)
    n_ge = jnp.where(is_found, float(_K), cnt_lo)
    overflow = jnp.any(n_ge > float(_K))

    @pl.when(jnp.logical_not(overflow))
    def _():
        o_ref[...] = jnp.where(key >= thr, jnp.maximum(x, 0.0), 0.0)

    @pl.when(overflow)
    def _():
        col = jax.lax.broadcasted_iota(jnp.int32, x.shape, 1)
        extra = n_ge.astype(jnp.int32) - _K
        tcol = jnp.where(key == thr, col, -1)
        cut = jnp.full((nrows, 1), jnp.iinfo(jnp.int32).max, jnp.int32)
        for _ in range(4):
            hi_col = jnp.max(jnp.where(tcol < cut, tcol, -1), axis=1, keepdims=True)
            cut = jnp.where(extra > 0, hi_col, cut)
            extra = jnp.maximum(extra - 1, 0)
        keep = (key > thr) | ((key == thr) & (col < cut))
        o_ref[...] = jnp.where(keep, jnp.maximum(x, 0.0), 0.0)


def kernel(x):
    m, n = x.shape
    grid = (m // _ROWS_PER_BLOCK,)
    return pl.pallas_call(
        _topk_mask_body,
        grid=grid,
        in_specs=[pl.BlockSpec((_ROWS_PER_BLOCK, n), lambda r: (r, 0))],
        out_specs=pl.BlockSpec((_ROWS_PER_BLOCK, n), lambda r: (r, 0)),
        out_shape=jax.ShapeDtypeStruct((m, n), x.dtype),
        compiler_params=pltpu.CompilerParams(
            dimension_semantics=("arbitrary",),
        ),
    )(x)


# speedup vs baseline: 7.6141x; 1.1518x over previous
import jax
import jax.numpy as jnp
from jax.experimental import pallas as pl
from jax.experimental.pallas import tpu as pltpu

_K = 64
_ROWS_PER_BLOCK = 16
_LANES = 128
_TOP_PER_LANE = 8
_IMIN = jnp.iinfo(jnp.int32).min


def _to_key(v):
    i = jax.lax.bitcast_convert_type(v, jnp.int32)
    return jnp.where(i >= 0, i, jnp.bitwise_xor(jnp.bitwise_not(i), jnp.int32(_IMIN)))


def _from_key(k):
    i = jnp.where(k >= 0, k, jnp.bitwise_not(jnp.bitwise_xor(k, jnp.int32(_IMIN))))
    return jax.lax.bitcast_convert_type(i, jnp.float32)


def _topk_mask_body(x_ref, o_ref):
    x = x_ref[...]
    nrows, ncols = x.shape
    nchunks = ncols // _LANES
    neg_inf = jnp.float32(-jnp.inf)

    chunks = [x[:, c * _LANES:(c + 1) * _LANES] for c in range(nchunks)]

    def tree_max(vals):
        while len(vals) > 1:
            nxt = [jnp.maximum(vals[j], vals[j + 1]) for j in range(0, len(vals) - 1, 2)]
            if len(vals) % 2:
                nxt.append(vals[-1])
            vals = nxt
        return vals[0]

    tops = [tree_max(chunks)]
    for _ in range(_TOP_PER_LANE - 1):
        prev = tops[-1]
        tops.append(tree_max([jnp.where(c < prev, c, neg_inf) for c in chunks]))
    cand = _to_key(jnp.concatenate(tops, axis=1))
    kmax = jnp.max(cand, axis=1, keepdims=True)

    def small_body(_, carry):
        glo, ghi = carry
        gmid = (glo >> 1) + (ghi >> 1) + (glo & ghi & 1)
        gcnt = jnp.sum(jnp.where(cand >= gmid, 1.0, 0.0), axis=1, keepdims=True)
        ge = gcnt >= float(_K)
        return jnp.where(ge, gmid, glo), jnp.where(ge, ghi, gmid)

    cmin = jnp.min(cand, axis=1, keepdims=True)
    tstar, _ = jax.lax.fori_loop(0, 32, small_body, (cmin, kmax + 1))

    def cnt_ge(t_key):
        t = _from_key(t_key)
        return jnp.sum(jnp.where(x >= t, 1.0, 0.0), axis=1, keepdims=True)

    def cond(carry):
        lo, hi, found, thr, cnt_lo = carry
        return jnp.any((found == 0) & ((hi - 1) > lo))

    def body(carry):
        lo, hi, found, thr, cnt_lo = carry
        mid = (lo >> 1) + (hi >> 1) + (lo & hi & 1)
        cnt = cnt_ge(mid)
        hit = cnt == float(_K)
        ge = cnt >= float(_K)
        thr = jnp.where(hit & (found == 0), mid, thr)
        found = found | hit.astype(jnp.int32)
        lo = jnp.where(ge, mid, lo)
        hi = jnp.where(ge, hi, mid)
        cnt_lo = jnp.where(ge, cnt, cnt_lo)
        return lo, hi, found, thr, cnt_lo

    cnt0 = cnt_ge(tstar)
    found0 = (cnt0 == float(_K)).astype(jnp.int32)
    carry0 = (tstar, kmax + 1, found0, tstar, cnt0)
    lo, hi, found, thr, cnt_lo = jax.lax.while_loop(cond, body, carry0)
    is_found = found == 1
    thr_f = _from_key(jnp.where(is_found, thr, lo))
    n_ge = jnp.where(is_found, float(_K), cnt_lo)
    overflow = jnp.any(n_ge > float(_K))

    @pl.when(jnp.logical_not(overflow))
    def _():
        o_ref[...] = jnp.where(x >= thr_f, jnp.maximum(x, 0.0), 0.0)

    @pl.when(overflow)
    def _():
        col = jax.lax.broadcasted_iota(jnp.int32, x.shape, 1)
        extra = n_ge.astype(jnp.int32) - _K
        tcol = jnp.where(x == thr_f, col, -1)
        cut = jnp.full((nrows, 1), jnp.iinfo(jnp.int32).max, jnp.int32)
        for _ in range(4):
            hi_col = jnp.max(jnp.where(tcol < cut, tcol, -1), axis=1, keepdims=True)
            cut = jnp.where(extra > 0, hi_col, cut)
            extra = jnp.maximum(extra - 1, 0)
        keep = (x > thr_f) | ((x == thr_f) & (col < cut))
        o_ref[...] = jnp.where(keep, jnp.maximum(x, 0.0), 0.0)


def kernel(x):
    m, n = x.shape
    grid = (m // _ROWS_PER_BLOCK,)
    return pl.pallas_call(
        _topk_mask_body,
        grid=grid,
        in_specs=[pl.BlockSpec((_ROWS_PER_BLOCK, n), lambda r: (r, 0))],
        out_specs=pl.BlockSpec((_ROWS_PER_BLOCK, n), lambda r: (r, 0)),
        out_shape=jax.ShapeDtypeStruct((m, n), x.dtype),
        compiler_params=pltpu.CompilerParams(
            dimension_semantics=("arbitrary",),
        ),
    )(x)


# speedup vs baseline: 7.6191x; 1.0007x over previous
import jax
import jax.numpy as jnp
from jax.experimental import pallas as pl
from jax.experimental.pallas import tpu as pltpu

_K = 64
_ROWS_PER_BLOCK = 16
_LANES = 128
_TOP_PER_LANE = 8
_IMIN = jnp.iinfo(jnp.int32).min


def _to_key(v):
    i = jax.lax.bitcast_convert_type(v, jnp.int32)
    return jnp.where(i >= 0, i, jnp.bitwise_xor(jnp.bitwise_not(i), jnp.int32(_IMIN)))


def _from_key(k):
    i = jnp.where(k >= 0, k, jnp.bitwise_not(jnp.bitwise_xor(k, jnp.int32(_IMIN))))
    return jax.lax.bitcast_convert_type(i, jnp.float32)


def _topk_mask_body(x_ref, o_ref):
    x = x_ref[...]
    nrows, ncols = x.shape
    nchunks = ncols // _LANES
    neg_inf = jnp.float32(-jnp.inf)

    chunks = [x[:, c * _LANES:(c + 1) * _LANES] for c in range(nchunks)]

    def tree_max(vals):
        while len(vals) > 1:
            nxt = [jnp.maximum(vals[j], vals[j + 1]) for j in range(0, len(vals) - 1, 2)]
            if len(vals) % 2:
                nxt.append(vals[-1])
            vals = nxt
        return vals[0]

    tops = [tree_max(chunks)]
    for _ in range(_TOP_PER_LANE - 1):
        prev = tops[-1]
        tops.append(tree_max([jnp.where(c < prev, c, neg_inf) for c in chunks]))
    cand = _to_key(jnp.concatenate(tops, axis=1))
    kmax = jnp.max(cand, axis=1, keepdims=True)

    def small_body(_, carry):
        glo, ghi = carry
        gmid = (glo >> 1) + (ghi >> 1) + (glo & ghi & 1)
        gcnt = jnp.sum(jnp.where(cand >= gmid, 1.0, 0.0), axis=1, keepdims=True)
        ge = gcnt >= float(_K)
        return jnp.where(ge, gmid, glo), jnp.where(ge, ghi, gmid)

    cmin = jnp.min(cand, axis=1, keepdims=True)
    tstar, _ = jax.lax.fori_loop(0, 32, small_body, (cmin, kmax + 1))

    def cnt_ge(t_key):
        t = _from_key(t_key)
        return jnp.sum(jnp.where(x >= t, 1.0, 0.0), axis=1, keepdims=True)

    def cond(carry):
        lo, hi, found, thr, cnt_lo = carry
        return jnp.any((found == 0) & ((hi - 1) > lo))

    def body(carry):
        lo, hi, found, thr, cnt_lo = carry
        mid = (lo >> 1) + (hi >> 1) + (lo & hi & 1)
        cnt = cnt_ge(mid)
        hit = cnt == float(_K)
        ge = cnt >= float(_K)
        thr = jnp.where(hit & (found == 0), mid, thr)
        found = found | hit.astype(jnp.int32)
        lo = jnp.where(ge, mid, lo)
        hi = jnp.where(ge, hi, mid)
        cnt_lo = jnp.where(ge, cnt, cnt_lo)
        return lo, hi, found, thr, cnt_lo

    cnt0 = cnt_ge(tstar)
    found0 = (cnt0 == float(_K)).astype(jnp.int32)
    carry0 = (tstar, kmax + 1, found0, tstar, cnt0)
    lo, hi, found, thr, cnt_lo = jax.lax.while_loop(cond, body, carry0)
    is_found = found == 1
    thr_f = _from_key(jnp.where(is_found, thr, lo))
    n_ge = jnp.where(is_found, float(_K), cnt_lo)
    overflow = jnp.any(n_ge > float(_K))

    @pl.when(jnp.logical_not(overflow))
    def _():
        o_ref[...] = jnp.where(x >= thr_f, jnp.maximum(x, 0.0), 0.0)

    @pl.when(overflow)
    def _():
        col = jax.lax.broadcasted_iota(jnp.int32, x.shape, 1)
        extra = n_ge.astype(jnp.int32) - _K
        tcol = jnp.where(x == thr_f, col, -1)
        cut = jnp.full((nrows, 1), jnp.iinfo(jnp.int32).max, jnp.int32)
        for _ in range(4):
            hi_col = jnp.max(jnp.where(tcol < cut, tcol, -1), axis=1, keepdims=True)
            cut = jnp.where(extra > 0, hi_col, cut)
            extra = jnp.maximum(extra - 1, 0)
        keep = (x > thr_f) | ((x == thr_f) & (col < cut))
        o_ref[...] = jnp.where(keep, jnp.maximum(x, 0.0), 0.0)


def kernel(x):
    m, n = x.shape
    grid = (m // _ROWS_PER_BLOCK,)
    return pl.pallas_call(
        _topk_mask_body,
        grid=grid,
        in_specs=[pl.BlockSpec((_ROWS_PER_BLOCK, n), lambda r: (r, 0))],
        out_specs=pl.BlockSpec((_ROWS_PER_BLOCK, n), lambda r: (r, 0)),
        out_shape=jax.ShapeDtypeStruct((m, n), x.dtype),
        compiler_params=pltpu.CompilerParams(
            dimension_semantics=("parallel",),
        ),
    )(x)


# speedup vs baseline: 10.1839x; 1.3366x over previous
import jax
import jax.numpy as jnp
from jax.experimental import pallas as pl
from jax.experimental.pallas import tpu as pltpu

_K = 64
_ROWS_PER_BLOCK = 32
_LANES = 128
_TOP_PER_LANE = 8
_IMIN = jnp.iinfo(jnp.int32).min


def _to_key(v):
    i = jax.lax.bitcast_convert_type(v, jnp.int32)
    return jnp.where(i >= 0, i, jnp.bitwise_xor(jnp.bitwise_not(i), jnp.int32(_IMIN)))


def _from_key(k):
    i = jnp.where(k >= 0, k, jnp.bitwise_not(jnp.bitwise_xor(k, jnp.int32(_IMIN))))
    return jax.lax.bitcast_convert_type(i, jnp.float32)


def _topk_mask_body(x_ref, o_ref):
    x = x_ref[...]
    nrows, ncols = x.shape
    nchunks = ncols // _LANES
    neg_inf = jnp.float32(-jnp.inf)

    chunks = [x[:, c * _LANES:(c + 1) * _LANES] for c in range(nchunks)]

    def tree_max(vals):
        while len(vals) > 1:
            nxt = [jnp.maximum(vals[j], vals[j + 1]) for j in range(0, len(vals) - 1, 2)]
            if len(vals) % 2:
                nxt.append(vals[-1])
            vals = nxt
        return vals[0]

    tops = [tree_max(chunks)]
    for _ in range(_TOP_PER_LANE - 1):
        prev = tops[-1]
        tops.append(tree_max([jnp.where(c < prev, c, neg_inf) for c in chunks]))
    cand = _to_key(jnp.concatenate(tops, axis=1))
    kmax = jnp.max(cand, axis=1, keepdims=True)

    def small_body(_, carry):
        glo, ghi = carry
        gmid = (glo >> 1) + (ghi >> 1) + (glo & ghi & 1)
        gcnt = jnp.sum(jnp.where(cand >= gmid, 1.0, 0.0), axis=1, keepdims=True)
        ge = gcnt >= float(_K)
        return jnp.where(ge, gmid, glo), jnp.where(ge, ghi, gmid)

    cmin = jnp.min(cand, axis=1, keepdims=True)
    tstar, _ = jax.lax.fori_loop(0, 32, small_body, (cmin, kmax + 1))

    def cnt_ge(t_key):
        t = _from_key(t_key)
        return jnp.sum(jnp.where(x >= t, 1.0, 0.0), axis=1, keepdims=True)

    def cond(carry):
        lo, hi, found, thr, cnt_lo = carry
        return jnp.any((found == 0) & ((hi - 1) > lo))

    def body(carry):
        lo, hi, found, thr, cnt_lo = carry
        mid = (lo >> 1) + (hi >> 1) + (lo & hi & 1)
        cnt = cnt_ge(mid)
        hit = cnt == float(_K)
        ge = cnt >= float(_K)
        thr = jnp.where(hit & (found == 0), mid, thr)
        found = found | hit.astype(jnp.int32)
        lo = jnp.where(ge, mid, lo)
        hi = jnp.where(ge, hi, mid)
        cnt_lo = jnp.where(ge, cnt, cnt_lo)
        return lo, hi, found, thr, cnt_lo

    cnt0 = cnt_ge(tstar)
    found0 = (cnt0 == float(_K)).astype(jnp.int32)
    carry0 = (tstar, kmax + 1, found0, tstar, cnt0)
    lo, hi, found, thr, cnt_lo = jax.lax.while_loop(cond, body, carry0)
    is_found = found == 1
    thr_f = _from_key(jnp.where(is_found, thr, lo))
    n_ge = jnp.where(is_found, float(_K), cnt_lo)
    overflow = jnp.any(n_ge > float(_K))

    @pl.when(jnp.logical_not(overflow))
    def _():
        o_ref[...] = jnp.where(x >= thr_f, jnp.maximum(x, 0.0), 0.0)

    @pl.when(overflow)
    def _():
        col = jax.lax.broadcasted_iota(jnp.int32, x.shape, 1)
        extra = n_ge.astype(jnp.int32) - _K
        tcol = jnp.where(x == thr_f, col, -1)
        cut = jnp.full((nrows, 1), jnp.iinfo(jnp.int32).max, jnp.int32)
        for _ in range(4):
            hi_col = jnp.max(jnp.where(tcol < cut, tcol, -1), axis=1, keepdims=True)
            cut = jnp.where(extra > 0, hi_col, cut)
            extra = jnp.maximum(extra - 1, 0)
        keep = (x > thr_f) | ((x == thr_f) & (col < cut))
        o_ref[...] = jnp.where(keep, jnp.maximum(x, 0.0), 0.0)


def kernel(x):
    m, n = x.shape
    grid = (m // _ROWS_PER_BLOCK,)
    return pl.pallas_call(
        _topk_mask_body,
        grid=grid,
        in_specs=[pl.BlockSpec((_ROWS_PER_BLOCK, n), lambda r: (r, 0))],
        out_specs=pl.BlockSpec((_ROWS_PER_BLOCK, n), lambda r: (r, 0)),
        out_shape=jax.ShapeDtypeStruct((m, n), x.dtype),
        compiler_params=pltpu.CompilerParams(
            dimension_semantics=("parallel",),
        ),
    )(x)


# speedup vs baseline: 12.1119x; 1.1893x over previous
import jax
import jax.numpy as jnp
from jax.experimental import pallas as pl
from jax.experimental.pallas import tpu as pltpu

_K = 64
_ROWS_PER_BLOCK = 64
_LANES = 128
_TOP_PER_LANE = 8
_IMIN = jnp.iinfo(jnp.int32).min


def _to_key(v):
    i = jax.lax.bitcast_convert_type(v, jnp.int32)
    return jnp.where(i >= 0, i, jnp.bitwise_xor(jnp.bitwise_not(i), jnp.int32(_IMIN)))


def _from_key(k):
    i = jnp.where(k >= 0, k, jnp.bitwise_not(jnp.bitwise_xor(k, jnp.int32(_IMIN))))
    return jax.lax.bitcast_convert_type(i, jnp.float32)


def _topk_mask_body(x_ref, o_ref):
    x = x_ref[...]
    nrows, ncols = x.shape
    nchunks = ncols // _LANES
    neg_inf = jnp.float32(-jnp.inf)

    chunks = [x[:, c * _LANES:(c + 1) * _LANES] for c in range(nchunks)]

    def tree_max(vals):
        while len(vals) > 1:
            nxt = [jnp.maximum(vals[j], vals[j + 1]) for j in range(0, len(vals) - 1, 2)]
            if len(vals) % 2:
                nxt.append(vals[-1])
            vals = nxt
        return vals[0]

    tops = [tree_max(chunks)]
    for _ in range(_TOP_PER_LANE - 1):
        prev = tops[-1]
        tops.append(tree_max([jnp.where(c < prev, c, neg_inf) for c in chunks]))
    cand = _to_key(jnp.concatenate(tops, axis=1))
    kmax = jnp.max(cand, axis=1, keepdims=True)

    def small_body(_, carry):
        glo, ghi = carry
        gmid = (glo >> 1) + (ghi >> 1) + (glo & ghi & 1)
        gcnt = jnp.sum(jnp.where(cand >= gmid, 1.0, 0.0), axis=1, keepdims=True)
        ge = gcnt >= float(_K)
        return jnp.where(ge, gmid, glo), jnp.where(ge, ghi, gmid)

    cmin = jnp.min(cand, axis=1, keepdims=True)
    tstar, _ = jax.lax.fori_loop(0, 32, small_body, (cmin, kmax + 1))

    def cnt_ge(t_key):
        t = _from_key(t_key)
        return jnp.sum(jnp.where(x >= t, 1.0, 0.0), axis=1, keepdims=True)

    def cond(carry):
        lo, hi, found, thr, cnt_lo = carry
        return jnp.any((found == 0) & ((hi - 1) > lo))

    def body(carry):
        lo, hi, found, thr, cnt_lo = carry
        mid = (lo >> 1) + (hi >> 1) + (lo & hi & 1)
        cnt = cnt_ge(mid)
        hit = cnt == float(_K)
        ge = cnt >= float(_K)
        thr = jnp.where(hit & (found == 0), mid, thr)
        found = found | hit.astype(jnp.int32)
        lo = jnp.where(ge, mid, lo)
        hi = jnp.where(ge, hi, mid)
        cnt_lo = jnp.where(ge, cnt, cnt_lo)
        return lo, hi, found, thr, cnt_lo

    cnt0 = cnt_ge(tstar)
    found0 = (cnt0 == float(_K)).astype(jnp.int32)
    carry0 = (tstar, kmax + 1, found0, tstar, cnt0)
    lo, hi, found, thr, cnt_lo = jax.lax.while_loop(cond, body, carry0)
    is_found = found == 1
    thr_f = _from_key(jnp.where(is_found, thr, lo))
    n_ge = jnp.where(is_found, float(_K), cnt_lo)
    overflow = jnp.any(n_ge > float(_K))

    @pl.when(jnp.logical_not(overflow))
    def _():
        o_ref[...] = jnp.where(x >= thr_f, jnp.maximum(x, 0.0), 0.0)

    @pl.when(overflow)
    def _():
        col = jax.lax.broadcasted_iota(jnp.int32, x.shape, 1)
        extra = n_ge.astype(jnp.int32) - _K
        tcol = jnp.where(x == thr_f, col, -1)
        cut = jnp.full((nrows, 1), jnp.iinfo(jnp.int32).max, jnp.int32)
        for _ in range(4):
            hi_col = jnp.max(jnp.where(tcol < cut, tcol, -1), axis=1, keepdims=True)
            cut = jnp.where(extra > 0, hi_col, cut)
            extra = jnp.maximum(extra - 1, 0)
        keep = (x > thr_f) | ((x == thr_f) & (col < cut))
        o_ref[...] = jnp.where(keep, jnp.maximum(x, 0.0), 0.0)


def kernel(x):
    m, n = x.shape
    grid = (m // _ROWS_PER_BLOCK,)
    return pl.pallas_call(
        _topk_mask_body,
        grid=grid,
        in_specs=[pl.BlockSpec((_ROWS_PER_BLOCK, n), lambda r: (r, 0))],
        out_specs=pl.BlockSpec((_ROWS_PER_BLOCK, n), lambda r: (r, 0)),
        out_shape=jax.ShapeDtypeStruct((m, n), x.dtype),
        compiler_params=pltpu.CompilerParams(
            dimension_semantics=("parallel",),
        ),
    )(x)


# speedup vs baseline: 16.9727x; 1.4013x over previous
import jax
import jax.numpy as jnp
from jax.experimental import pallas as pl
from jax.experimental.pallas import tpu as pltpu

_K = 64
_ROWS_PER_BLOCK = 64
_LANES = 128
_TOP_PER_LANE = 5
_IMIN = jnp.iinfo(jnp.int32).min


def _to_key(v):
    i = jax.lax.bitcast_convert_type(v, jnp.int32)
    return jnp.where(i >= 0, i, jnp.bitwise_xor(jnp.bitwise_not(i), jnp.int32(_IMIN)))


def _from_key(k):
    i = jnp.where(k >= 0, k, jnp.bitwise_not(jnp.bitwise_xor(k, jnp.int32(_IMIN))))
    return jax.lax.bitcast_convert_type(i, jnp.float32)


def _topk_mask_body(x_ref, o_ref):
    x = x_ref[...]
    nrows, ncols = x.shape
    nchunks = ncols // _LANES
    neg_inf = jnp.float32(-jnp.inf)

    chunks = [x[:, c * _LANES:(c + 1) * _LANES] for c in range(nchunks)]

    def tree_max(vals):
        while len(vals) > 1:
            nxt = [jnp.maximum(vals[j], vals[j + 1]) for j in range(0, len(vals) - 1, 2)]
            if len(vals) % 2:
                nxt.append(vals[-1])
            vals = nxt
        return vals[0]

    fams = [chunks[: nchunks // 2], chunks[nchunks // 2:]]
    tops = []
    prevs = [None, None]
    for _ in range(_TOP_PER_LANE):
        for f in range(2):
            if prevs[f] is None:
                cur = tree_max(fams[f])
            else:
                cur = tree_max([jnp.where(c < prevs[f], c, neg_inf) for c in fams[f]])
            prevs[f] = cur
            tops.append(cur)
    cand = _to_key(jnp.concatenate(tops, axis=1))
    kmax = jnp.max(cand, axis=1, keepdims=True)

    def small_cond(carry):
        glo, ghi, gfound, gthr = carry
        return jnp.any((gfound == 0) & ((ghi - 1) > glo))

    def small_body(carry):
        glo, ghi, gfound, gthr = carry
        gmid = (glo >> 1) + (ghi >> 1) + (glo & ghi & 1)
        gcnt = jnp.sum(jnp.where(cand >= gmid, 1.0, 0.0), axis=1, keepdims=True)
        hit = gcnt == float(_K)
        ge = gcnt >= float(_K)
        gthr = jnp.where(hit & (gfound == 0), gmid, gthr)
        gfound = gfound | hit.astype(jnp.int32)
        glo = jnp.where(ge, gmid, glo)
        ghi = jnp.where(ge, ghi, gmid)
        return glo, ghi, gfound, gthr

    cmin = jnp.min(cand, axis=1, keepdims=True)
    glo, _, gfound, gthr = jax.lax.while_loop(
        small_cond, small_body,
        (cmin, kmax + 1, jnp.zeros((nrows, 1), jnp.int32), cmin))
    tstar = jnp.where(gfound == 1, gthr, glo)

    def cnt_ge(t_key):
        t = _from_key(t_key)
        return jnp.sum(jnp.where(x >= t, 1.0, 0.0), axis=1, keepdims=True)

    def cond(carry):
        lo, hi, found, thr, cnt_lo = carry
        return jnp.any((found == 0) & ((hi - 1) > lo))

    def body(carry):
        lo, hi, found, thr, cnt_lo = carry
        mid = (lo >> 1) + (hi >> 1) + (lo & hi & 1)
        cnt = cnt_ge(mid)
        hit = cnt == float(_K)
        ge = cnt >= float(_K)
        thr = jnp.where(hit & (found == 0), mid, thr)
        found = found | hit.astype(jnp.int32)
        lo = jnp.where(ge, mid, lo)
        hi = jnp.where(ge, hi, mid)
        cnt_lo = jnp.where(ge, cnt, cnt_lo)
        return lo, hi, found, thr, cnt_lo

    cnt0 = cnt_ge(tstar)
    found0 = (cnt0 == float(_K)).astype(jnp.int32)
    carry0 = (tstar, kmax + 1, found0, tstar, cnt0)
    lo, hi, found, thr, cnt_lo = jax.lax.while_loop(cond, body, carry0)
    is_found = found == 1
    thr_f = _from_key(jnp.where(is_found, thr, lo))
    n_ge = jnp.where(is_found, float(_K), cnt_lo)
    overflow = jnp.any(n_ge > float(_K))

    @pl.when(jnp.logical_not(overflow))
    def _():
        o_ref[...] = jnp.where(x >= thr_f, jnp.maximum(x, 0.0), 0.0)

    @pl.when(overflow)
    def _():
        col = jax.lax.broadcasted_iota(jnp.int32, x.shape, 1)
        extra = n_ge.astype(jnp.int32) - _K
        tcol = jnp.where(x == thr_f, col, -1)
        cut = jnp.full((nrows, 1), jnp.iinfo(jnp.int32).max, jnp.int32)
        for _ in range(4):
            hi_col = jnp.max(jnp.where(tcol < cut, tcol, -1), axis=1, keepdims=True)
            cut = jnp.where(extra > 0, hi_col, cut)
            extra = jnp.maximum(extra - 1, 0)
        keep = (x > thr_f) | ((x == thr_f) & (col < cut))
        o_ref[...] = jnp.where(keep, jnp.maximum(x, 0.0), 0.0)


def kernel(x):
    m, n = x.shape
    grid = (m // _ROWS_PER_BLOCK,)
    return pl.pallas_call(
        _topk_mask_body,
        grid=grid,
        in_specs=[pl.BlockSpec((_ROWS_PER_BLOCK, n), lambda r: (r, 0))],
        out_specs=pl.BlockSpec((_ROWS_PER_BLOCK, n), lambda r: (r, 0)),
        out_shape=jax.ShapeDtypeStruct((m, n), x.dtype),
        compiler_params=pltpu.CompilerParams(
            dimension_semantics=("parallel",),
        ),
    )(x)


# speedup vs baseline: 18.9154x; 1.1145x over previous
import jax
import jax.numpy as jnp
from jax.experimental import pallas as pl
from jax.experimental.pallas import tpu as pltpu

_K = 64
_ROWS_PER_BLOCK = 64
_LANES = 128
_TOP_PER_LANE = 4
_IMIN = jnp.iinfo(jnp.int32).min


def _to_key(v):
    i = jax.lax.bitcast_convert_type(v, jnp.int32)
    return jnp.where(i >= 0, i, jnp.bitwise_xor(jnp.bitwise_not(i), jnp.int32(_IMIN)))


def _from_key(k):
    i = jnp.where(k >= 0, k, jnp.bitwise_not(jnp.bitwise_xor(k, jnp.int32(_IMIN))))
    return jax.lax.bitcast_convert_type(i, jnp.float32)


def _topk_mask_body(x_ref, o_ref):
    x = x_ref[...]
    nrows, ncols = x.shape
    nchunks = ncols // _LANES
    neg_inf = jnp.float32(-jnp.inf)

    chunks = [x[:, c * _LANES:(c + 1) * _LANES] for c in range(nchunks)]

    def tree_max(vals):
        while len(vals) > 1:
            nxt = [jnp.maximum(vals[j], vals[j + 1]) for j in range(0, len(vals) - 1, 2)]
            if len(vals) % 2:
                nxt.append(vals[-1])
            vals = nxt
        return vals[0]

    nf = nchunks // 4
    fams = [chunks[i * nf:(i + 1) * nf] for i in range(4)]
    tops = []
    prevs = [None, None, None, None]
    for _ in range(_TOP_PER_LANE):
        for f in range(4):
            if prevs[f] is None:
                cur = tree_max(fams[f])
            else:
                cur = tree_max([jnp.where(c < prevs[f], c, neg_inf) for c in fams[f]])
            prevs[f] = cur
            tops.append(cur)
    cand = _to_key(jnp.concatenate(tops, axis=1))
    kmax = jnp.max(cand, axis=1, keepdims=True)

    def small_cond(carry):
        glo, ghi, gfound, gthr = carry
        return jnp.any((gfound == 0) & ((ghi - 1) > glo))

    def small_body(carry):
        glo, ghi, gfound, gthr = carry
        gmid = (glo >> 1) + (ghi >> 1) + (glo & ghi & 1)
        gcnt = jnp.sum(jnp.where(cand >= gmid, 1.0, 0.0), axis=1, keepdims=True)
        hit = gcnt == float(_K)
        ge = gcnt >= float(_K)
        gthr = jnp.where(hit & (gfound == 0), gmid, gthr)
        gfound = gfound | hit.astype(jnp.int32)
        glo = jnp.where(ge, gmid, glo)
        ghi = jnp.where(ge, ghi, gmid)
        return glo, ghi, gfound, gthr

    cmin = jnp.min(cand, axis=1, keepdims=True)
    glo, _, gfound, gthr = jax.lax.while_loop(
        small_cond, small_body,
        (cmin, kmax + 1, jnp.zeros((nrows, 1), jnp.int32), cmin))
    tstar = jnp.where(gfound == 1, gthr, glo)

    def cnt_ge(t_key):
        t = _from_key(t_key)
        return jnp.sum(jnp.where(x >= t, 1.0, 0.0), axis=1, keepdims=True)

    def cond(carry):
        lo, hi, found, thr, cnt_lo = carry
        return jnp.any((found == 0) & ((hi - 1) > lo))

    def body(carry):
        lo, hi, found, thr, cnt_lo = carry
        mid = (lo >> 1) + (hi >> 1) + (lo & hi & 1)
        cnt = cnt_ge(mid)
        hit = cnt == float(_K)
        ge = cnt >= float(_K)
        thr = jnp.where(hit & (found == 0), mid, thr)
        found = found | hit.astype(jnp.int32)
        lo = jnp.where(ge, mid, lo)
        hi = jnp.where(ge, hi, mid)
        cnt_lo = jnp.where(ge, cnt, cnt_lo)
        return lo, hi, found, thr, cnt_lo

    cnt0 = cnt_ge(tstar)
    found0 = (cnt0 == float(_K)).astype(jnp.int32)
    carry0 = (tstar, kmax + 1, found0, tstar, cnt0)
    lo, hi, found, thr, cnt_lo = jax.lax.while_loop(cond, body, carry0)
    is_found = found == 1
    thr_f = _from_key(jnp.where(is_found, thr, lo))
    n_ge = jnp.where(is_found, float(_K), cnt_lo)
    overflow = jnp.any(n_ge > float(_K))

    @pl.when(jnp.logical_not(overflow))
    def _():
        o_ref[...] = jnp.where(x >= thr_f, jnp.maximum(x, 0.0), 0.0)

    @pl.when(overflow)
    def _():
        col = jax.lax.broadcasted_iota(jnp.int32, x.shape, 1)
        extra = n_ge.astype(jnp.int32) - _K
        tcol = jnp.where(x == thr_f, col, -1)
        cut = jnp.full((nrows, 1), jnp.iinfo(jnp.int32).max, jnp.int32)
        for _ in range(4):
            hi_col = jnp.max(jnp.where(tcol < cut, tcol, -1), axis=1, keepdims=True)
            cut = jnp.where(extra > 0, hi_col, cut)
            extra = jnp.maximum(extra - 1, 0)
        keep = (x > thr_f) | ((x == thr_f) & (col < cut))
        o_ref[...] = jnp.where(keep, jnp.maximum(x, 0.0), 0.0)


def kernel(x):
    m, n = x.shape
    grid = (m // _ROWS_PER_BLOCK,)
    return pl.pallas_call(
        _topk_mask_body,
        grid=grid,
        in_specs=[pl.BlockSpec((_ROWS_PER_BLOCK, n), lambda r: (r, 0))],
        out_specs=pl.BlockSpec((_ROWS_PER_BLOCK, n), lambda r: (r, 0)),
        out_shape=jax.ShapeDtypeStruct((m, n), x.dtype),
        compiler_params=pltpu.CompilerParams(
            dimension_semantics=("parallel",),
        ),
    )(x)
